# baseline (speedup 1.0000x reference)
import functools

import jax
import jax.numpy as jnp
from jax import lax
from jax.experimental import pallas as pl
from jax.experimental.pallas import tpu as pltpu
from jax.experimental.pallas import tpu_sc as plsc

_B = 10000
_K = 32
_D = 128

_NC = 2
_NS = 16
_CH = 128

_CH0 = 168
_CH1 = 0
_CT = _NS * (_CH0 + _CH1)
_N_PAD = _CT * _CH

_BB = 200
_GRID = _B // _BB
_UBLK = (_B * _K) // _BB


def _sc_gather_body(table_h, idx_h, out_h, idx_v, bufs, gsems, osems):
    c = lax.axis_index("c")
    s = lax.axis_index("s")

    def start_g(j, b):
        pltpu.make_async_copy(
            table_h.at[idx_v.at[j]], bufs.at[b], gsems.at[b]).start()

    def wait_g(b):
        pltpu.make_async_copy(
            table_h.at[idx_v.at[0]], bufs.at[b], gsems.at[b]).wait()

    def start_s(row0, b):
        pltpu.make_async_copy(
            bufs.at[b], out_h.at[pl.ds(row0, _CH)], osems.at[b]).start()

    def wait_s(b):
        pltpu.make_async_copy(
            bufs.at[b], out_h.at[pl.ds(0, _CH)], osems.at[b]).wait()

    def run(nch, base_chunk):
        pltpu.sync_copy(idx_h.at[pl.ds(base_chunk, nch)],
                        idx_v.at[pl.ds(0, nch)])
        base_row = base_chunk * _CH
        nsuper = nch // 4

        start_g(0, 0)
        start_g(1, 1)

        @pl.loop(0, nsuper)
        def _super(u):
            p0 = 4 * u
            p1 = 4 * u + 2
            wait_g(0)
            wait_g(1)

            @pl.when(u > 0)
            def _():
                wait_s(2)
                wait_s(3)

            start_g(p1, 2)
            start_g(p1 + 1, 3)
            start_s(base_row + p0 * _CH, 0)
            start_s(base_row + (p0 + 1) * _CH, 1)

            wait_g(2)
            wait_g(3)
            wait_s(0)
            wait_s(1)

            @pl.when(u < nsuper - 1)
            def _():
                start_g(p0 + 4, 0)
                start_g(p0 + 5, 1)

            start_s(base_row + p1 * _CH, 2)
            start_s(base_row + (p1 + 1) * _CH, 3)

        wait_s(2)
        wait_s(3)

    @pl.when(c == 0)
    def _core0():
        run(_CH0, s * _CH0)

    if _CH1 > 0:
        @pl.when(c == 1)
        def _core1():
            run(_CH1, _NS * _CH0 + s * _CH1)


@jax.jit
def _sc_gather(table, idx):
    mesh = plsc.VectorSubcoreMesh(core_axis_name="c", subcore_axis_name="s")
    k = pl.kernel(
        _sc_gather_body,
        out_type=jax.ShapeDtypeStruct((_N_PAD, _D), jnp.float32),
        mesh=mesh,
        scratch_types=[
            pltpu.VMEM((_CH0, _CH), jnp.int32),
            pltpu.VMEM((4, _CH, _D), jnp.float32),
            pltpu.SemaphoreType.DMA((4,)),
            pltpu.SemaphoreType.DMA((4,)),
        ],
    )
    return k(table, idx)


def _tc_mlp_body(e3_ref, u_ref, w1t_ref, w1b_ref, w2_ref, w3t_ref,
                 b1_ref, b2_ref, b3_ref, out_ref):
    e3 = e3_ref[...]
    e2 = e3.reshape(_BB * _K, _D)
    u = u_ref[...]

    uw = jnp.dot(u, w1b_ref[...], preferred_element_type=jnp.float32)
    uw = uw + b1_ref[...]
    z1 = jnp.dot(e2, w1t_ref[...], preferred_element_type=jnp.float32)
    h1 = jnp.maximum(z1.reshape(_BB, _K, _D) + uw[:, None, :], 0.0)

    h2 = jnp.dot(h1.reshape(_BB * _K, _D), w2_ref[...],
                 preferred_element_type=jnp.float32)
    h2 = jnp.maximum(h2 + b2_ref[...], 0.0)

    w3row = w3t_ref[...].reshape(1, 1, _D)
    t = jnp.sum(h2.reshape(_BB, _K, _D) * w3row, axis=2, keepdims=True)
    t = t + b3_ref[0, 0]

    m = jnp.max(t, axis=1, keepdims=True)
    p = jnp.exp(t - m)
    s = jnp.sum(p, axis=1, keepdims=True)
    att = p / s

    out_ref[...] = jnp.sum(e3 * att, axis=1)


def _tc_mlp(e3, u, w1t, w1b, w2, w3t, b1, b2, b3):
    return pl.pallas_call(
        _tc_mlp_body,
        grid=(_GRID,),
        in_specs=[
            pl.BlockSpec((_BB, _K, _D), lambda i: (i, 0, 0)),
            pl.BlockSpec((_BB, _D), lambda i: (i + _UBLK, 0)),
            pl.BlockSpec((_D, _D), lambda i: (0, 0)),
            pl.BlockSpec((_D, _D), lambda i: (0, 0)),
            pl.BlockSpec((_D, _D), lambda i: (0, 0)),
            pl.BlockSpec((1, _D), lambda i: (0, 0)),
            pl.BlockSpec((1, _D), lambda i: (0, 0)),
            pl.BlockSpec((1, _D), lambda i: (0, 0)),
            pl.BlockSpec((1, 1), lambda i: (0, 0)),
        ],
        out_specs=pl.BlockSpec((_BB, _D), lambda i: (i, 0)),
        out_shape=jax.ShapeDtypeStruct((_B, _D), jnp.float32),
    )(e3, u, w1t, w1b, w2, w3t, b1, b2, b3)


def kernel(nodes, to_neighs, u2e, W1, b1, W2, b2, W3, b3):
    idx = jnp.zeros((_N_PAD,), jnp.int32)
    idx = idx.at[: _B * _K].set(to_neighs.reshape(-1))
    idx = idx.at[_B * _K: _B * _K + _B].set(nodes)
    idx = idx.reshape(_CT, _CH)

    rows = _sc_gather(u2e, idx)
    e3 = rows.reshape(_N_PAD // _K, _K, _D)

    return _tc_mlp(e3, rows, W1[:_D], W1[_D:], W2, W3.reshape(1, _D),
                   b1.reshape(1, _D), b2.reshape(1, _D), b3.reshape(1, 1))


# speedup vs baseline: 1.0881x; 1.0881x over previous
import functools

import jax
import jax.numpy as jnp
from jax import lax
from jax.experimental import pallas as pl
from jax.experimental.pallas import tpu as pltpu
from jax.experimental.pallas import tpu_sc as plsc

_B = 10000
_K = 32
_D = 128

_NC = 2
_NS = 16
_CH = 128

_CH0 = 168
_CH1 = 0
_CT = _NS * (_CH0 + _CH1)
_N_PAD = _CT * _CH

_BB = 200
_GRID = _B // _BB
_UBLK = (_B * _K) // _BB


def _sc_gather_body(table_h, idx_h, out_h, idx_v, bufs, gsems, osems):
    c = lax.axis_index("c")
    s = lax.axis_index("s")

    def start_g(j, b):
        pltpu.make_async_copy(
            table_h.at[idx_v.at[j]], bufs.at[b], gsems.at[b]).start()

    def wait_g(b):
        pltpu.make_async_copy(
            table_h.at[idx_v.at[0]], bufs.at[b], gsems.at[b]).wait()

    def start_s(row0, b):
        pltpu.make_async_copy(
            bufs.at[b], out_h.at[pl.ds(row0, _CH)], osems.at[b]).start()

    def wait_s(b):
        pltpu.make_async_copy(
            bufs.at[b], out_h.at[pl.ds(0, _CH)], osems.at[b]).wait()

    def run(nch, base_chunk):
        pltpu.sync_copy(idx_h.at[pl.ds(base_chunk, nch)],
                        idx_v.at[pl.ds(0, nch)])
        base_row = base_chunk * _CH

        for b in range(4):
            start_g(b, b)

        @pl.loop(0, nch // 6)
        def _round(t):
            for b in range(6):
                j = 6 * t + b
                wait_g(b)
                start_s(base_row + j * _CH, b)
                b2 = (b + 4) % 6

                @pl.when(j + 4 < nch)
                def _():
                    @pl.when(j >= 2)
                    def _():
                        wait_s(b2)

                    start_g(j + 4, b2)

        for b in range(6):
            wait_s(b)

    @pl.when(c == 0)
    def _core0():
        run(_CH0, s * _CH0)

    if _CH1 > 0:
        @pl.when(c == 1)
        def _core1():
            run(_CH1, _NS * _CH0 + s * _CH1)


@jax.jit
def _sc_gather(table, idx):
    mesh = plsc.VectorSubcoreMesh(core_axis_name="c", subcore_axis_name="s")
    k = pl.kernel(
        _sc_gather_body,
        out_type=jax.ShapeDtypeStruct((_N_PAD, _D), jnp.float32),
        mesh=mesh,
        scratch_types=[
            pltpu.VMEM((_CH0, _CH), jnp.int32),
            pltpu.VMEM((6, _CH, _D), jnp.float32),
            pltpu.SemaphoreType.DMA((6,)),
            pltpu.SemaphoreType.DMA((6,)),
        ],
    )
    return k(table, idx)


def _tc_mlp_body(e3_ref, u_ref, w1t_ref, w1b_ref, w2_ref, w3t_ref,
                 b1_ref, b2_ref, b3_ref, out_ref):
    e3 = e3_ref[...]
    e2 = e3.reshape(_BB * _K, _D)
    u = u_ref[...]

    uw = jnp.dot(u, w1b_ref[...], preferred_element_type=jnp.float32)
    uw = uw + b1_ref[...]
    z1 = jnp.dot(e2, w1t_ref[...], preferred_element_type=jnp.float32)
    h1 = jnp.maximum(z1.reshape(_BB, _K, _D) + uw[:, None, :], 0.0)

    h2 = jnp.dot(h1.reshape(_BB * _K, _D), w2_ref[...],
                 preferred_element_type=jnp.float32)
    h2 = jnp.maximum(h2 + b2_ref[...], 0.0)

    w3row = w3t_ref[...].reshape(1, 1, _D)
    t = jnp.sum(h2.reshape(_BB, _K, _D) * w3row, axis=2, keepdims=True)
    t = t + b3_ref[0, 0]

    m = jnp.max(t, axis=1, keepdims=True)
    p = jnp.exp(t - m)
    s = jnp.sum(p, axis=1, keepdims=True)
    att = p / s

    out_ref[...] = jnp.sum(e3 * att, axis=1)


def _tc_mlp(e3, u, w1t, w1b, w2, w3t, b1, b2, b3):
    return pl.pallas_call(
        _tc_mlp_body,
        grid=(_GRID,),
        in_specs=[
            pl.BlockSpec((_BB, _K, _D), lambda i: (i, 0, 0)),
            pl.BlockSpec((_BB, _D), lambda i: (i + _UBLK, 0)),
            pl.BlockSpec((_D, _D), lambda i: (0, 0)),
            pl.BlockSpec((_D, _D), lambda i: (0, 0)),
            pl.BlockSpec((_D, _D), lambda i: (0, 0)),
            pl.BlockSpec((1, _D), lambda i: (0, 0)),
            pl.BlockSpec((1, _D), lambda i: (0, 0)),
            pl.BlockSpec((1, _D), lambda i: (0, 0)),
            pl.BlockSpec((1, 1), lambda i: (0, 0)),
        ],
        out_specs=pl.BlockSpec((_BB, _D), lambda i: (i, 0)),
        out_shape=jax.ShapeDtypeStruct((_B, _D), jnp.float32),
    )(e3, u, w1t, w1b, w2, w3t, b1, b2, b3)


def kernel(nodes, to_neighs, u2e, W1, b1, W2, b2, W3, b3):
    idx = jnp.zeros((_N_PAD,), jnp.int32)
    idx = idx.at[: _B * _K].set(to_neighs.reshape(-1))
    idx = idx.at[_B * _K: _B * _K + _B].set(nodes)
    idx = idx.reshape(_CT, _CH)

    rows = _sc_gather(u2e, idx)
    e3 = rows.reshape(_N_PAD // _K, _K, _D)

    return _tc_mlp(e3, rows, W1[:_D], W1[_D:], W2, W3.reshape(1, _D),
                   b1.reshape(1, _D), b2.reshape(1, _D), b3.reshape(1, 1))


# speedup vs baseline: 1.1026x; 1.0133x over previous
import functools

import jax
import jax.numpy as jnp
from jax import lax
from jax.experimental import pallas as pl
from jax.experimental.pallas import tpu as pltpu
from jax.experimental.pallas import tpu_sc as plsc

_B = 10000
_K = 32
_D = 128
_DW = _D // 2

_NC = 2
_NS = 16
_NW = _NC * _NS
_CH = 128

_CPW = 84
_CT = _NW * _CPW
_N_PAD = _CT * _CH

_BB = 200
_GRID = _B // _BB
_UBLK = (_B * _K) // _BB


def _sc_gather_body(table_h, idx_h, out_h, idx_v, bufs, gsems, osems):
    wid = lax.axis_index("s") * _NC + lax.axis_index("c")

    def start_g(j, b):
        pltpu.make_async_copy(
            table_h.at[idx_v.at[j]], bufs.at[b], gsems.at[b]).start()

    def wait_g(b):
        pltpu.make_async_copy(
            table_h.at[idx_v.at[0]], bufs.at[b], gsems.at[b]).wait()

    def start_s(row0, b):
        pltpu.make_async_copy(
            bufs.at[b], out_h.at[pl.ds(row0, _CH)], osems.at[b]).start()

    def wait_s(b):
        pltpu.make_async_copy(
            bufs.at[b], out_h.at[pl.ds(0, _CH)], osems.at[b]).wait()

    pltpu.sync_copy(idx_h.at[wid], idx_v)
    base_row = wid * _CPW * _CH

    for b in range(4):
        start_g(b, b)

    @pl.loop(0, _CPW // 6)
    def _round(t):
        for b in range(6):
            j = 6 * t + b
            wait_g(b)
            start_s(base_row + j * _CH, b)
            b2 = (b + 4) % 6

            @pl.when(j + 4 < _CPW)
            def _():
                @pl.when(j >= 2)
                def _():
                    wait_s(b2)

                start_g(j + 4, b2)

    for b in range(6):
        wait_s(b)


@jax.jit
def _sc_gather(table, idx):
    mesh = plsc.VectorSubcoreMesh(core_axis_name="c", subcore_axis_name="s")
    k = pl.kernel(
        _sc_gather_body,
        out_type=jax.ShapeDtypeStruct((_N_PAD, _D), jnp.float32),
        mesh=mesh,
        scratch_types=[
            pltpu.VMEM((_CPW, _CH), jnp.int32),
            pltpu.VMEM((6, _CH, _D), jnp.float32),
            pltpu.SemaphoreType.DMA((6,)),
            pltpu.SemaphoreType.DMA((6,)),
        ],
    )
    return k(table, idx)


def _tc_mlp_body(e3_ref, u_ref, w1t_ref, w1b_ref, w2_ref, w3t_ref,
                 b1_ref, b2_ref, b3_ref, out_ref):
    e3 = e3_ref[...]
    e2 = e3.reshape(_BB * _K, _D).astype(jnp.bfloat16)
    u = u_ref[...].astype(jnp.bfloat16)

    uw = jnp.dot(u, w1b_ref[...], preferred_element_type=jnp.float32)
    uw = uw + b1_ref[...]
    z1 = jnp.dot(e2, w1t_ref[...], preferred_element_type=jnp.float32)
    h1 = jnp.maximum(z1.reshape(_BB, _K, _D) + uw[:, None, :], 0.0)

    h2 = jnp.dot(h1.reshape(_BB * _K, _D).astype(jnp.bfloat16), w2_ref[...],
                 preferred_element_type=jnp.float32)
    h2 = jnp.maximum(h2 + b2_ref[...], 0.0)

    w3row = w3t_ref[...].reshape(1, 1, _D)
    t = jnp.sum(h2.reshape(_BB, _K, _D) * w3row, axis=2, keepdims=True)
    t = t + b3_ref[0, 0]

    m = jnp.max(t, axis=1, keepdims=True)
    p = jnp.exp(t - m)
    s = jnp.sum(p, axis=1, keepdims=True)
    att = p / s

    out_ref[...] = jnp.sum(e3 * att, axis=1)


def _tc_mlp(e3, u, w1t, w1b, w2, w3t, b1, b2, b3):
    return pl.pallas_call(
        _tc_mlp_body,
        grid=(_GRID,),
        in_specs=[
            pl.BlockSpec((_BB, _K, _D), lambda i: (i, 0, 0)),
            pl.BlockSpec((_BB, _D), lambda i: (i + _UBLK, 0)),
            pl.BlockSpec((_D, _D), lambda i: (0, 0)),
            pl.BlockSpec((_D, _D), lambda i: (0, 0)),
            pl.BlockSpec((_D, _D), lambda i: (0, 0)),
            pl.BlockSpec((1, _D), lambda i: (0, 0)),
            pl.BlockSpec((1, _D), lambda i: (0, 0)),
            pl.BlockSpec((1, _D), lambda i: (0, 0)),
            pl.BlockSpec((1, 1), lambda i: (0, 0)),
        ],
        out_specs=pl.BlockSpec((_BB, _D), lambda i: (i, 0)),
        out_shape=jax.ShapeDtypeStruct((_B, _D), jnp.float32),
    )(e3, u, w1t, w1b, w2, w3t, b1, b2, b3)


def kernel(nodes, to_neighs, u2e, W1, b1, W2, b2, W3, b3):
    idx = jnp.zeros((_N_PAD,), jnp.int32)
    idx = idx.at[: _B * _K].set(to_neighs.reshape(-1))
    idx = idx.at[_B * _K: _B * _K + _B].set(nodes)
    idx = idx.reshape(_NW, _CPW, _CH)

    rows = _sc_gather(u2e, idx)
    e3 = rows.reshape(_N_PAD // _K, _K, _D)

    bf = jnp.bfloat16
    return _tc_mlp(e3, rows, W1[:_D].astype(bf), W1[_D:].astype(bf),
                   W2.astype(bf), W3.reshape(1, _D),
                   b1.reshape(1, _D), b2.reshape(1, _D), b3.reshape(1, 1))


# speedup vs baseline: 1.3368x; 1.2124x over previous
import functools

import jax
import jax.numpy as jnp
from jax import lax
from jax.experimental import pallas as pl
from jax.experimental.pallas import tpu as pltpu
from jax.experimental.pallas import tpu_sc as plsc

_B = 10000
_K = 32
_D = 128

_NC = 2
_NS = 16
_NW = _NC * _NS
_CH = 128

_C1 = 80
_N1_PAD = _NW * _C1 * _CH
_C2 = 3
_N2_PAD = _NW * _C2 * _CH

_BB = 200
_GRID = _B // _BB


def _sc_gather_body(table_h, idx1_h, idx2_h, out1_h, out2_h,
                    idx1_v, idx2_v, bufs, gsems, osems):
    wid = lax.axis_index("s") * _NC + lax.axis_index("c")
    pltpu.sync_copy(idx1_h.at[wid], idx1_v)
    pltpu.sync_copy(idx2_h.at[wid], idx2_v)

    def start_g(idx_v, j, b):
        pltpu.make_async_copy(
            table_h.at[idx_v.at[j]], bufs.at[b], gsems.at[b]).start()

    def wait_g(b):
        pltpu.make_async_copy(
            table_h.at[idx1_v.at[0]], bufs.at[b], gsems.at[b]).wait()

    def start_s(out_h, row0, b):
        pltpu.make_async_copy(
            bufs.at[b], out_h.at[pl.ds(row0, _CH)], osems.at[b]).start()

    def wait_s(b):
        pltpu.make_async_copy(
            bufs.at[b], out1_h.at[pl.ds(0, _CH)], osems.at[b]).wait()

    base1 = wid * _C1 * _CH

    for b in range(3):
        start_g(idx1_v, b, b)

    @pl.loop(0, _C1 // 5)
    def _round(t):
        for b in range(5):
            j = 5 * t + b
            wait_g(b)
            start_s(out1_h, base1 + j * _CH, b)
            b2 = (b + 3) % 5

            @pl.when(j + 3 < _C1)
            def _():
                @pl.when(j >= 2)
                def _():
                    wait_s(b2)

                start_g(idx1_v, j + 3, b2)

    for b in range(5):
        wait_s(b)

    base2 = wid * _C2 * _CH
    for j in range(_C2):
        pltpu.async_copy(table_h.at[idx2_v.at[j]], bufs.at[j], gsems.at[j]).wait()
        start_s(out2_h, base2 + j * _CH, j)
    for j in range(_C2):
        pltpu.make_async_copy(
            bufs.at[j], out2_h.at[pl.ds(0, _CH)], osems.at[j]).wait()


@jax.jit
def _sc_gather(table, idx1, idx2):
    mesh = plsc.VectorSubcoreMesh(core_axis_name="c", subcore_axis_name="s")
    k = pl.kernel(
        _sc_gather_body,
        out_type=(
            jax.ShapeDtypeStruct((_N1_PAD, _D), jnp.float32),
            jax.ShapeDtypeStruct((_N2_PAD, _D), jnp.float32),
        ),
        mesh=mesh,
        scratch_types=[
            pltpu.VMEM((_C1, _CH), jnp.int32),
            pltpu.VMEM((_C2, _CH), jnp.int32),
            pltpu.VMEM((5, _CH, _D), jnp.float32),
            pltpu.SemaphoreType.DMA((5,)),
            pltpu.SemaphoreType.DMA((5,)),
        ],
    )
    return k(table, idx1, idx2)


def _tc_mlp_body(e3_ref, u_ref, w1t_ref, w1b_ref, w2_ref, w3t_ref,
                 b1_ref, b2_ref, b3_ref, out_ref):
    e3 = e3_ref[...]
    e2 = e3.reshape(_BB * _K, _D).astype(jnp.bfloat16)
    u = u_ref[...].astype(jnp.bfloat16)

    uw = jnp.dot(u, w1b_ref[...], preferred_element_type=jnp.float32)
    uw = uw + b1_ref[...]
    z1 = jnp.dot(e2, w1t_ref[...], preferred_element_type=jnp.float32)
    h1 = jnp.maximum(z1.reshape(_BB, _K, _D) + uw[:, None, :], 0.0)

    h2 = jnp.dot(h1.reshape(_BB * _K, _D).astype(jnp.bfloat16), w2_ref[...],
                 preferred_element_type=jnp.float32)
    h2 = jnp.maximum(h2 + b2_ref[...], 0.0)

    w3row = w3t_ref[...].reshape(1, 1, _D)
    t = jnp.sum(h2.reshape(_BB, _K, _D) * w3row, axis=2, keepdims=True)
    t = t + b3_ref[0, 0]

    m = jnp.max(t, axis=1, keepdims=True)
    p = jnp.exp(t - m)
    s = jnp.sum(p, axis=1, keepdims=True)
    att = p / s

    out_ref[...] = jnp.sum(e3 * att, axis=1)


def _tc_mlp(e3, u, w1t, w1b, w2, w3t, b1, b2, b3):
    return pl.pallas_call(
        _tc_mlp_body,
        grid=(_GRID,),
        in_specs=[
            pl.BlockSpec((_BB, _K, _D), lambda i: (i, 0, 0)),
            pl.BlockSpec((_BB, _D), lambda i: (i, 0)),
            pl.BlockSpec((_D, _D), lambda i: (0, 0)),
            pl.BlockSpec((_D, _D), lambda i: (0, 0)),
            pl.BlockSpec((_D, _D), lambda i: (0, 0)),
            pl.BlockSpec((1, _D), lambda i: (0, 0)),
            pl.BlockSpec((1, _D), lambda i: (0, 0)),
            pl.BlockSpec((1, _D), lambda i: (0, 0)),
            pl.BlockSpec((1, 1), lambda i: (0, 0)),
        ],
        out_specs=pl.BlockSpec((_BB, _D), lambda i: (i, 0)),
        out_shape=jax.ShapeDtypeStruct((_B, _D), jnp.float32),
    )(e3, u, w1t, w1b, w2, w3t, b1, b2, b3)


def kernel(nodes, to_neighs, u2e, W1, b1, W2, b2, W3, b3):
    idx1 = jnp.zeros((_N1_PAD,), jnp.int32).at[: _B * _K].set(
        to_neighs.reshape(-1)).reshape(_NW, _C1, _CH)
    idx2 = jnp.zeros((_N2_PAD,), jnp.int32).at[:_B].set(
        nodes).reshape(_NW, _C2, _CH)

    e_rows, u_rows = _sc_gather(u2e, idx1, idx2)
    e3 = e_rows.reshape(_N1_PAD // _K, _K, _D)

    bf = jnp.bfloat16
    return _tc_mlp(e3, u_rows, W1[:_D].astype(bf), W1[_D:].astype(bf),
                   W2.astype(bf), W3.reshape(1, _D),
                   b1.reshape(1, _D), b2.reshape(1, _D), b3.reshape(1, 1))


# speedup vs baseline: 1.3958x; 1.0441x over previous
import functools

import jax
import jax.numpy as jnp
from jax import lax
from jax.experimental import pallas as pl
from jax.experimental.pallas import tpu as pltpu
from jax.experimental.pallas import tpu_sc as plsc

_B = 10000
_K = 32
_D = 128

_NC = 2
_NS = 16
_NW = _NC * _NS
_CH = 128

_C1 = 40
_N1_PAD = _NW * _C1 * _CH
_C2 = 3
_N2_PAD = _NW * _C2 * _CH

_BB = 200
_BH = _B // 2
_GRID = _BH // _BB


def _sc_gather_body(with_self, table_h, idx1_h, idx2_h, out1_h, out2_h,
                    idx1_v, idx2_v, bufs, gsems, osems):
    wid = lax.axis_index("s") * _NC + lax.axis_index("c")
    pltpu.sync_copy(idx1_h.at[wid], idx1_v)
    if with_self:
        pltpu.sync_copy(idx2_h.at[wid], idx2_v)

    def start_g(idx_v, j, b):
        pltpu.make_async_copy(
            table_h.at[idx_v.at[j]], bufs.at[b], gsems.at[b]).start()

    def wait_g(b):
        pltpu.make_async_copy(
            table_h.at[idx1_v.at[0]], bufs.at[b], gsems.at[b]).wait()

    def start_s(out_h, row0, b):
        pltpu.make_async_copy(
            bufs.at[b], out_h.at[pl.ds(row0, _CH)], osems.at[b]).start()

    def wait_s(b):
        pltpu.make_async_copy(
            bufs.at[b], out1_h.at[pl.ds(0, _CH)], osems.at[b]).wait()

    base1 = wid * _C1 * _CH

    for b in range(3):
        start_g(idx1_v, b, b)

    @pl.loop(0, _C1 // 5)
    def _round(t):
        for b in range(5):
            j = 5 * t + b
            wait_g(b)
            start_s(out1_h, base1 + j * _CH, b)
            b2 = (b + 3) % 5

            @pl.when(j + 3 < _C1)
            def _():
                @pl.when(j >= 2)
                def _():
                    wait_s(b2)

                start_g(idx1_v, j + 3, b2)

    for b in range(5):
        wait_s(b)

    if with_self:
        base2 = wid * _C2 * _CH
        for j in range(_C2):
            pltpu.async_copy(
                table_h.at[idx2_v.at[j]], bufs.at[j], gsems.at[j]).wait()
            start_s(out2_h, base2 + j * _CH, j)
        for j in range(_C2):
            pltpu.make_async_copy(
                bufs.at[j], out2_h.at[pl.ds(0, _CH)], osems.at[j]).wait()


_SCRATCH = [
    pltpu.VMEM((_C1, _CH), jnp.int32),
    pltpu.VMEM((_C2, _CH), jnp.int32),
    pltpu.VMEM((5, _CH, _D), jnp.float32),
    pltpu.SemaphoreType.DMA((5,)),
    pltpu.SemaphoreType.DMA((5,)),
]
_MESH = plsc.VectorSubcoreMesh(core_axis_name="c", subcore_axis_name="s")


@jax.jit
def _sc_gather_a(table, idx1, idx2):
    def body(table_h, idx1_h, idx2_h, out1_h, out2_h, *scratch):
        _sc_gather_body(True, table_h, idx1_h, idx2_h, out1_h, out2_h,
                        *scratch)

    k = pl.kernel(
        body,
        out_type=(
            jax.ShapeDtypeStruct((_N1_PAD, _D), jnp.float32),
            jax.ShapeDtypeStruct((_N2_PAD, _D), jnp.float32),
        ),
        mesh=_MESH,
        scratch_types=_SCRATCH,
    )
    return k(table, idx1, idx2)


@jax.jit
def _sc_gather_b(table, idx1):
    def body(table_h, idx1_h, out1_h, *scratch):
        idx1_v, idx2_v, bufs, gsems, osems = scratch
        _sc_gather_body(False, table_h, idx1_h, None, out1_h, None,
                        idx1_v, idx2_v, bufs, gsems, osems)

    k = pl.kernel(
        body,
        out_type=jax.ShapeDtypeStruct((_N1_PAD, _D), jnp.float32),
        mesh=_MESH,
        scratch_types=_SCRATCH,
    )
    return k(table, idx1)


def _tc_mlp_body(e3_ref, u_ref, w1t_ref, w1b_ref, w2_ref, w3t_ref,
                 b1_ref, b2_ref, b3_ref, out_ref):
    e3 = e3_ref[...]
    e2 = e3.reshape(_BB * _K, _D).astype(jnp.bfloat16)
    u = u_ref[...].astype(jnp.bfloat16)

    uw = jnp.dot(u, w1b_ref[...], preferred_element_type=jnp.float32)
    uw = uw + b1_ref[...]
    z1 = jnp.dot(e2, w1t_ref[...], preferred_element_type=jnp.float32)
    h1 = jnp.maximum(z1.reshape(_BB, _K, _D) + uw[:, None, :], 0.0)

    h2 = jnp.dot(h1.reshape(_BB * _K, _D).astype(jnp.bfloat16), w2_ref[...],
                 preferred_element_type=jnp.float32)
    h2 = jnp.maximum(h2 + b2_ref[...], 0.0)

    w3row = w3t_ref[...].reshape(1, 1, _D)
    t = jnp.sum(h2.reshape(_BB, _K, _D) * w3row, axis=2, keepdims=True)
    t = t + b3_ref[0, 0]

    m = jnp.max(t, axis=1, keepdims=True)
    p = jnp.exp(t - m)
    s = jnp.sum(p, axis=1, keepdims=True)
    att = p / s

    out_ref[...] = jnp.sum(e3 * att, axis=1)


def _tc_mlp(half, e3, u, w1t, w1b, w2, w3t, b1, b2, b3):
    ublk = half * (_BH // _BB)
    return pl.pallas_call(
        _tc_mlp_body,
        grid=(_GRID,),
        in_specs=[
            pl.BlockSpec((_BB, _K, _D), lambda i: (i, 0, 0)),
            pl.BlockSpec((_BB, _D), lambda i: (i + ublk, 0)),
            pl.BlockSpec((_D, _D), lambda i: (0, 0)),
            pl.BlockSpec((_D, _D), lambda i: (0, 0)),
            pl.BlockSpec((_D, _D), lambda i: (0, 0)),
            pl.BlockSpec((1, _D), lambda i: (0, 0)),
            pl.BlockSpec((1, _D), lambda i: (0, 0)),
            pl.BlockSpec((1, _D), lambda i: (0, 0)),
            pl.BlockSpec((1, 1), lambda i: (0, 0)),
        ],
        out_specs=pl.BlockSpec((_BB, _D), lambda i: (i, 0)),
        out_shape=jax.ShapeDtypeStruct((_BH, _D), jnp.float32),
    )(e3, u, w1t, w1b, w2, w3t, b1, b2, b3)


def kernel(nodes, to_neighs, u2e, W1, b1, W2, b2, W3, b3):
    nflat = to_neighs.reshape(-1)
    idx1a = jnp.zeros((_N1_PAD,), jnp.int32).at[: _BH * _K].set(
        nflat[: _BH * _K]).reshape(_NW, _C1, _CH)
    idx1b = jnp.zeros((_N1_PAD,), jnp.int32).at[: _BH * _K].set(
        nflat[_BH * _K:]).reshape(_NW, _C1, _CH)
    idx2 = jnp.zeros((_N2_PAD,), jnp.int32).at[:_B].set(
        nodes).reshape(_NW, _C2, _CH)

    ea, u_rows = _sc_gather_a(u2e, idx1a, idx2)
    eb = _sc_gather_b(u2e, idx1b)

    bf = jnp.bfloat16
    args = (W1[:_D].astype(bf), W1[_D:].astype(bf), W2.astype(bf),
            W3.reshape(1, _D), b1.reshape(1, _D), b2.reshape(1, _D),
            b3.reshape(1, 1))
    oa = _tc_mlp(0, ea.reshape(_N1_PAD // _K, _K, _D), u_rows, *args)
    ob = _tc_mlp(1, eb.reshape(_N1_PAD // _K, _K, _D), u_rows, *args)
    return jnp.concatenate([oa, ob], axis=0)


# speedup vs baseline: 1.5505x; 1.1108x over previous
import functools

import jax
import jax.numpy as jnp
from jax import lax
from jax.experimental import pallas as pl
from jax.experimental.pallas import tpu as pltpu
from jax.experimental.pallas import tpu_sc as plsc

_B = 10000
_K = 32
_D = 128

_NC = 2
_NS = 16
_NW = _NC * _NS
_CH = 128

_PARTS = ((3200, 25), (3200, 25), (3600, 30))
_C2 = 3
_N2_PAD = _NW * _C2 * _CH

_BB = 200


def _sc_gather_body(c1, with_self, table_h, idx1_h, idx2_h, out1_h, out2_h,
                    idx1_v, idx2_v, bufs, gsems, osems):
    wid = lax.axis_index("s") * _NC + lax.axis_index("c")
    pltpu.sync_copy(idx1_h.at[wid], idx1_v)
    if with_self:
        pltpu.sync_copy(idx2_h.at[wid], idx2_v)

    def start_g(idx_v, j, b):
        pltpu.make_async_copy(
            table_h.at[idx_v.at[j]], bufs.at[b], gsems.at[b]).start()

    def wait_g(b):
        pltpu.make_async_copy(
            table_h.at[idx1_v.at[0]], bufs.at[b], gsems.at[b]).wait()

    def start_s(out_h, row0, b):
        pltpu.make_async_copy(
            bufs.at[b], out_h.at[pl.ds(row0, _CH)], osems.at[b]).start()

    def wait_s(b):
        pltpu.make_async_copy(
            bufs.at[b], out1_h.at[pl.ds(0, _CH)], osems.at[b]).wait()

    base1 = wid * c1 * _CH

    for b in range(3):
        start_g(idx1_v, b, b)

    @pl.loop(0, c1 // 5)
    def _round(t):
        for b in range(5):
            j = 5 * t + b
            wait_g(b)
            start_s(out1_h, base1 + j * _CH, b)
            b2 = (b + 3) % 5

            @pl.when(j + 3 < c1)
            def _():
                @pl.when(j >= 2)
                def _():
                    wait_s(b2)

                start_g(idx1_v, j + 3, b2)

    for b in range(5):
        wait_s(b)

    if with_self:
        base2 = wid * _C2 * _CH
        for j in range(_C2):
            pltpu.async_copy(
                table_h.at[idx2_v.at[j]], bufs.at[j], gsems.at[j]).wait()
            start_s(out2_h, base2 + j * _CH, j)
        for j in range(_C2):
            pltpu.make_async_copy(
                bufs.at[j], out2_h.at[pl.ds(0, _CH)], osems.at[j]).wait()


_MESH = plsc.VectorSubcoreMesh(core_axis_name="c", subcore_axis_name="s")


def _scratch(c1):
    return [
        pltpu.VMEM((c1, _CH), jnp.int32),
        pltpu.VMEM((_C2, _CH), jnp.int32),
        pltpu.VMEM((5, _CH, _D), jnp.float32),
        pltpu.SemaphoreType.DMA((5,)),
        pltpu.SemaphoreType.DMA((5,)),
    ]


def _make_sc(c1, with_self):
    n1 = _NW * c1 * _CH

    if with_self:
        def body(table_h, idx1_h, idx2_h, out1_h, out2_h, *scratch):
            _sc_gather_body(c1, True, table_h, idx1_h, idx2_h,
                            out1_h, out2_h, *scratch)

        out_type = (jax.ShapeDtypeStruct((n1, _D), jnp.float32),
                    jax.ShapeDtypeStruct((_N2_PAD, _D), jnp.float32))
    else:
        def body(table_h, idx1_h, out1_h, *scratch):
            idx1_v, idx2_v, bufs, gsems, osems = scratch
            _sc_gather_body(c1, False, table_h, idx1_h, None, out1_h, None,
                            idx1_v, idx2_v, bufs, gsems, osems)

        out_type = jax.ShapeDtypeStruct((n1, _D), jnp.float32)

    k = pl.kernel(body, out_type=out_type, mesh=_MESH,
                  scratch_types=_scratch(c1))
    return k


_SC_A = _make_sc(_PARTS[0][1], True)
_SC_B = _make_sc(_PARTS[1][1], False)
_SC_C = _make_sc(_PARTS[2][1], False)


def _tc_mlp_body(e3_ref, u_ref, w1t_ref, w1b_ref, w2_ref, w3t_ref,
                 b1_ref, b2_ref, b3_ref, out_ref):
    e3 = e3_ref[...]
    e2 = e3.reshape(_BB * _K, _D).astype(jnp.bfloat16)
    u = u_ref[...].astype(jnp.bfloat16)

    uw = jnp.dot(u, w1b_ref[...], preferred_element_type=jnp.float32)
    uw = uw + b1_ref[...]
    z1 = jnp.dot(e2, w1t_ref[...], preferred_element_type=jnp.float32)
    h1 = jnp.maximum(z1.reshape(_BB, _K, _D) + uw[:, None, :], 0.0)

    h2 = jnp.dot(h1.reshape(_BB * _K, _D).astype(jnp.bfloat16), w2_ref[...],
                 preferred_element_type=jnp.float32)
    h2 = jnp.maximum(h2 + b2_ref[...], 0.0)

    w3row = w3t_ref[...].reshape(1, 1, _D)
    t = jnp.sum(h2.reshape(_BB, _K, _D) * w3row, axis=2, keepdims=True)
    t = t + b3_ref[0, 0]

    m = jnp.max(t, axis=1, keepdims=True)
    p = jnp.exp(t - m)
    s = jnp.sum(p, axis=1, keepdims=True)
    att = p / s

    out_ref[...] = jnp.sum(e3 * att, axis=1)


def _tc_mlp(nodes_p, ublk, e3, u, w1t, w1b, w2, w3t, b1, b2, b3):
    return pl.pallas_call(
        _tc_mlp_body,
        grid=(nodes_p // _BB,),
        in_specs=[
            pl.BlockSpec((_BB, _K, _D), lambda i: (i, 0, 0)),
            pl.BlockSpec((_BB, _D), lambda i: (i + ublk, 0)),
            pl.BlockSpec((_D, _D), lambda i: (0, 0)),
            pl.BlockSpec((_D, _D), lambda i: (0, 0)),
            pl.BlockSpec((_D, _D), lambda i: (0, 0)),
            pl.BlockSpec((1, _D), lambda i: (0, 0)),
            pl.BlockSpec((1, _D), lambda i: (0, 0)),
            pl.BlockSpec((1, _D), lambda i: (0, 0)),
            pl.BlockSpec((1, 1), lambda i: (0, 0)),
        ],
        out_specs=pl.BlockSpec((_BB, _D), lambda i: (i, 0)),
        out_shape=jax.ShapeDtypeStruct((nodes_p, _D), jnp.float32),
    )(e3, u, w1t, w1b, w2, w3t, b1, b2, b3)


def kernel(nodes, to_neighs, u2e, W1, b1, W2, b2, W3, b3):
    nflat = to_neighs.reshape(-1)
    idx2 = jnp.zeros((_N2_PAD,), jnp.int32).at[:_B].set(
        nodes).reshape(_NW, _C2, _CH)

    idx1 = []
    row0 = 0
    for nodes_p, c1 in _PARTS:
        n1 = _NW * c1 * _CH
        nrows = nodes_p * _K
        idx1.append(jnp.zeros((n1,), jnp.int32).at[:nrows].set(
            nflat[row0: row0 + nrows]).reshape(_NW, c1, _CH))
        row0 += nrows

    ea, u_rows = _SC_A(u2e, idx1[0], idx2)
    eb = _SC_B(u2e, idx1[1])
    ec = _SC_C(u2e, idx1[2])

    bf = jnp.bfloat16
    args = (W1[:_D].astype(bf), W1[_D:].astype(bf), W2.astype(bf),
            W3.reshape(1, _D), b1.reshape(1, _D), b2.reshape(1, _D),
            b3.reshape(1, 1))
    outs = []
    ublk = 0
    for (nodes_p, c1), rows in zip(_PARTS, (ea, eb, ec)):
        e3 = rows.reshape(rows.shape[0] // _K, _K, _D)
        outs.append(_tc_mlp(nodes_p, ublk, e3, u_rows, *args))
        ublk += nodes_p // _BB
    return jnp.concatenate(outs, axis=0)
